# baseline (speedup 1.0000x reference)
import functools

import jax
import jax.numpy as jnp
from jax import lax
from jax.experimental import pallas as pl
from jax.experimental.pallas import tpu as pltpu
from jax.experimental.pallas import tpu_sc as plsc

BATCH = 4096
EMB = 64
NC = 2
NS = 16
NW = NC * NS
BPW = BATCH // NW


def _sc_gather(user_emb, item_emb, beta_flat, user_id, last_item, pos_item,
               neg_item):
    mesh = plsc.VectorSubcoreMesh(core_axis_name="c", subcore_axis_name="s")

    @functools.partial(
        pl.kernel,
        mesh=mesh,
        out_type=(
            jax.ShapeDtypeStruct((BATCH, EMB), jnp.float32),
            jax.ShapeDtypeStruct((BATCH, EMB), jnp.float32),
            jax.ShapeDtypeStruct((BATCH, EMB), jnp.float32),
            jax.ShapeDtypeStruct((BATCH, EMB), jnp.float32),
            jax.ShapeDtypeStruct((BATCH, 1), jnp.float32),
            jax.ShapeDtypeStruct((BATCH, 1), jnp.float32),
        ),
        scratch_types=[
            pltpu.VMEM((BPW,), jnp.int32),
            pltpu.VMEM((BPW,), jnp.int32),
            pltpu.VMEM((BPW,), jnp.int32),
            pltpu.VMEM((BPW,), jnp.int32),
            pltpu.VMEM((BPW, EMB), jnp.float32),
            pltpu.VMEM((BPW, EMB), jnp.float32),
            pltpu.VMEM((BPW, EMB), jnp.float32),
            pltpu.VMEM((BPW, EMB), jnp.float32),
            pltpu.VMEM((BPW, 1), jnp.float32),
            pltpu.VMEM((BPW, 1), jnp.float32),
            pltpu.SemaphoreType.DMA,
        ],
    )
    def k(ue_hbm, ie_hbm, be_hbm, uid_hbm, li_hbm, pi_hbm, ni_hbm,
          gu_hbm, gl_hbm, gp_hbm, gn_hbm, bp_hbm, bn_hbm,
          iu_v, il_v, ip_v, in_v, ru_v, rl_v, rp_v, rn_v, bp_v, bn_v, sem):
        wid = lax.axis_index("s") * NC + lax.axis_index("c")
        base = wid * BPW
        pltpu.sync_copy(uid_hbm.at[pl.ds(base, BPW)], iu_v)
        pltpu.sync_copy(li_hbm.at[pl.ds(base, BPW)], il_v)
        pltpu.sync_copy(pi_hbm.at[pl.ds(base, BPW)], ip_v)
        pltpu.sync_copy(ni_hbm.at[pl.ds(base, BPW)], in_v)

        def body(c, carry):
            off = c * 16
            vu = iu_v[pl.ds(off, 16)]
            vl = il_v[pl.ds(off, 16)]
            vp = ip_v[pl.ds(off, 16)]
            vn = in_v[pl.ds(off, 16)]
            for j in range(16):
                r = off + j
                pltpu.async_copy(ue_hbm.at[pl.ds(vu[j], 1)],
                                 ru_v.at[pl.ds(r, 1)], sem)
                pltpu.async_copy(ie_hbm.at[pl.ds(vl[j], 1)],
                                 rl_v.at[pl.ds(r, 1)], sem)
                pltpu.async_copy(ie_hbm.at[pl.ds(vp[j], 1)],
                                 rp_v.at[pl.ds(r, 1)], sem)
                pltpu.async_copy(ie_hbm.at[pl.ds(vn[j], 1)],
                                 rn_v.at[pl.ds(r, 1)], sem)
                pltpu.async_copy(be_hbm.at[pl.ds(vp[j], 1)],
                                 bp_v.at[pl.ds(r, 1)], sem)
                pltpu.async_copy(be_hbm.at[pl.ds(vn[j], 1)],
                                 bn_v.at[pl.ds(r, 1)], sem)
            return carry

        lax.fori_loop(0, BPW // 16, body, 0)
        pltpu.make_async_copy(ue_hbm.at[pl.ds(0, BPW)], ru_v, sem).wait()
        pltpu.make_async_copy(ie_hbm.at[pl.ds(0, BPW)], rl_v, sem).wait()
        pltpu.make_async_copy(ie_hbm.at[pl.ds(0, BPW)], rp_v, sem).wait()
        pltpu.make_async_copy(ie_hbm.at[pl.ds(0, BPW)], rn_v, sem).wait()
        pltpu.make_async_copy(be_hbm.at[pl.ds(0, BPW)], bp_v, sem).wait()
        pltpu.make_async_copy(be_hbm.at[pl.ds(0, BPW)], bn_v, sem).wait()

        pltpu.sync_copy(ru_v, gu_hbm.at[pl.ds(base, BPW)])
        pltpu.sync_copy(rl_v, gl_hbm.at[pl.ds(base, BPW)])
        pltpu.sync_copy(rp_v, gp_hbm.at[pl.ds(base, BPW)])
        pltpu.sync_copy(rn_v, gn_hbm.at[pl.ds(base, BPW)])
        pltpu.sync_copy(bp_v, bp_hbm.at[pl.ds(base, BPW)])
        pltpu.sync_copy(bn_v, bn_hbm.at[pl.ds(base, BPW)])

    return k(user_emb, item_emb, beta_flat, user_id, last_item, pos_item,
             neg_item)


def _tc_body(gu_ref, gl_ref, gp_ref, gn_ref, bp_ref, bn_ref, t_ref,
             pos_ref, neg_ref):
    def scale(x):
        n2 = jnp.sum(x * x, axis=1, keepdims=True)
        return 1.0 / jnp.maximum(jnp.sqrt(n2), 1.0)

    gl = gl_ref[...]
    out = gu_ref[...] + t_ref[...] + gl * scale(gl)
    gp = gp_ref[...]
    dp = out - gp * scale(gp)
    gn = gn_ref[...]
    dn = out - gn * scale(gn)
    pos_ref[...] = bp_ref[...] - jnp.sqrt(jnp.sum(dp * dp))
    neg_ref[...] = bn_ref[...] - jnp.sqrt(jnp.sum(dn * dn))


def kernel(user_id, last_item, pos_item, neg_item, user_emb, item_emb, Beta,
           T):
    gu, gl, gp, gn, bp, bn = _sc_gather(
        user_emb, item_emb, Beta,
        user_id.astype(jnp.int32), last_item.astype(jnp.int32),
        pos_item.astype(jnp.int32), neg_item.astype(jnp.int32))
    pos_score, neg_score = pl.pallas_call(
        _tc_body,
        out_shape=(
            jax.ShapeDtypeStruct((BATCH, 1), jnp.float32),
            jax.ShapeDtypeStruct((BATCH, 1), jnp.float32),
        ),
    )(gu, gl, gp, gn, bp, bn, T)
    return (pos_score, neg_score)


# speedup vs baseline: 1.5818x; 1.5818x over previous
import functools

import jax
import jax.numpy as jnp
from jax import lax
from jax.experimental import pallas as pl
from jax.experimental.pallas import tpu as pltpu
from jax.experimental.pallas import tpu_sc as plsc

BATCH = 4096
EMB = 64
ITEMS = 100000
NC = 2
NS = 16
CPS = EMB // NC
BPT = BATCH // NS
NCH = BPT // 128


def _sc_gather(ue_t, ie_t, beta_flat, user_id, last_item, pos_item,
               neg_item):
    mesh = plsc.VectorSubcoreMesh(core_axis_name="c", subcore_axis_name="s")

    @functools.partial(
        pl.kernel,
        mesh=mesh,
        out_type=(
            jax.ShapeDtypeStruct((EMB, BATCH), jnp.float32),
            jax.ShapeDtypeStruct((EMB, BATCH), jnp.float32),
            jax.ShapeDtypeStruct((EMB, BATCH), jnp.float32),
            jax.ShapeDtypeStruct((EMB, BATCH), jnp.float32),
            jax.ShapeDtypeStruct((1, BATCH), jnp.float32),
            jax.ShapeDtypeStruct((1, BATCH), jnp.float32),
        ),
        scratch_types=[
            pltpu.VMEM_SHARED((ITEMS,), jnp.float32),
            pltpu.VMEM_SHARED((ITEMS,), jnp.float32),
            pltpu.VMEM_SHARED((ITEMS,), jnp.float32),
            pltpu.VMEM_SHARED((ITEMS,), jnp.float32),
            pltpu.VMEM_SHARED((ITEMS,), jnp.float32),
            pltpu.VMEM((BPT,), jnp.int32),
            pltpu.VMEM((BPT,), jnp.int32),
            pltpu.VMEM((BPT,), jnp.int32),
            pltpu.VMEM((BPT,), jnp.int32),
            pltpu.VMEM((CPS, BPT), jnp.float32),
            pltpu.VMEM((CPS, BPT), jnp.float32),
            pltpu.VMEM((CPS, BPT), jnp.float32),
            pltpu.VMEM((CPS, BPT), jnp.float32),
            pltpu.VMEM((BPT,), jnp.float32),
            pltpu.VMEM((BPT,), jnp.float32),
            pltpu.SemaphoreType.DMA,
            pltpu.SemaphoreType.DMA,
        ],
    )
    def k(ue_hbm, ie_hbm, be_hbm, uid_hbm, li_hbm, pi_hbm, ni_hbm,
          gu_hbm, gl_hbm, gp_hbm, gn_hbm, bp_hbm, bn_hbm,
          urow0, urow1, irow0, irow1, brow,
          iu_v, il_v, ip_v, in_v, ou_v, ol_v, op_v, on_v, bp_v, bn_v,
          sem_s, sem_g):
        cid = lax.axis_index("c")
        sid = lax.axis_index("s")
        cbase = cid * CPS
        bbase = sid * BPT
        urows = [urow0, urow1]
        irows = [irow0, irow1]

        pltpu.sync_copy(uid_hbm.at[pl.ds(bbase, BPT)], iu_v)
        pltpu.sync_copy(li_hbm.at[pl.ds(bbase, BPT)], il_v)
        pltpu.sync_copy(pi_hbm.at[pl.ds(bbase, BPT)], ip_v)
        pltpu.sync_copy(ni_hbm.at[pl.ds(bbase, BPT)], in_v)

        @pl.when(sid == 0)
        def _():
            pltpu.async_copy(ue_hbm.at[cbase], urows[0], sem_s)

        @pl.when(sid == 1)
        def _():
            pltpu.async_copy(ie_hbm.at[cbase], irows[0], sem_s)

        @pl.when(sid == 2)
        def _():
            pltpu.async_copy(be_hbm, brow, sem_s).wait()

        for cc in range(CPS):
            p = cc % 2
            @pl.when(sid == 0)
            def _():
                pltpu.make_async_copy(ue_hbm.at[cbase], urows[p],
                                      sem_s).wait()

            @pl.when(sid == 1)
            def _():
                pltpu.make_async_copy(ie_hbm.at[cbase], irows[p],
                                      sem_s).wait()

            plsc.subcore_barrier()
            if cc + 1 < CPS:
                @pl.when(sid == 0)
                def _():
                    pltpu.async_copy(ue_hbm.at[cbase + cc + 1],
                                     urows[1 - p], sem_s)

                @pl.when(sid == 1)
                def _():
                    pltpu.async_copy(ie_hbm.at[cbase + cc + 1],
                                     irows[1 - p], sem_s)

            cps = []
            for ch in range(NCH):
                sl = pl.ds(ch * 128, 128)
                cps.append(pltpu.async_copy(
                    urows[p].at[iu_v.at[sl]], ou_v.at[cc, sl], sem_g))
                cps.append(pltpu.async_copy(
                    irows[p].at[il_v.at[sl]], ol_v.at[cc, sl], sem_g))
                cps.append(pltpu.async_copy(
                    irows[p].at[ip_v.at[sl]], op_v.at[cc, sl], sem_g))
                cps.append(pltpu.async_copy(
                    irows[p].at[in_v.at[sl]], on_v.at[cc, sl], sem_g))
            for c in cps:
                c.wait()
            plsc.subcore_barrier()

        @pl.when(cid == 0)
        def _():
            cbs = []
            for ch in range(NCH):
                sl = pl.ds(ch * 128, 128)
                cbs.append(pltpu.async_copy(
                    brow.at[ip_v.at[sl]], bp_v.at[sl], sem_g))
                cbs.append(pltpu.async_copy(
                    brow.at[in_v.at[sl]], bn_v.at[sl], sem_g))
            for c in cbs:
                c.wait()
            pltpu.sync_copy(bp_v, bp_hbm.at[0, pl.ds(bbase, BPT)])
            pltpu.sync_copy(bn_v, bn_hbm.at[0, pl.ds(bbase, BPT)])

        pltpu.sync_copy(ou_v, gu_hbm.at[pl.ds(cbase, CPS), pl.ds(bbase, BPT)])
        pltpu.sync_copy(ol_v, gl_hbm.at[pl.ds(cbase, CPS), pl.ds(bbase, BPT)])
        pltpu.sync_copy(op_v, gp_hbm.at[pl.ds(cbase, CPS), pl.ds(bbase, BPT)])
        pltpu.sync_copy(on_v, gn_hbm.at[pl.ds(cbase, CPS), pl.ds(bbase, BPT)])

    return k(ue_t, ie_t, beta_flat, user_id, last_item, pos_item, neg_item)


def _tc_body(gu_ref, gl_ref, gp_ref, gn_ref, bp_ref, bn_ref, t_ref,
             pos_ref, neg_ref):
    def scale(x):
        n2 = jnp.sum(x * x, axis=0, keepdims=True)
        return 1.0 / jnp.maximum(jnp.sqrt(n2), 1.0)

    gl = gl_ref[...]
    out = gu_ref[...] + t_ref[...] + gl * scale(gl)
    gp = gp_ref[...]
    dp = out - gp * scale(gp)
    gn = gn_ref[...]
    dn = out - gn * scale(gn)
    pos_ref[...] = bp_ref[...] - jnp.sqrt(jnp.sum(dp * dp))
    neg_ref[...] = bn_ref[...] - jnp.sqrt(jnp.sum(dn * dn))


def kernel(user_id, last_item, pos_item, neg_item, user_emb, item_emb, Beta,
           T):
    gu, gl, gp, gn, bp, bn = _sc_gather(
        user_emb.T, item_emb.T, Beta.reshape(-1),
        user_id.astype(jnp.int32), last_item.astype(jnp.int32),
        pos_item.astype(jnp.int32), neg_item.astype(jnp.int32))
    pos_score, neg_score = pl.pallas_call(
        _tc_body,
        out_shape=(
            jax.ShapeDtypeStruct((1, BATCH), jnp.float32),
            jax.ShapeDtypeStruct((1, BATCH), jnp.float32),
        ),
    )(gu, gl, gp, gn, bp, bn, T.reshape(EMB, 1))
    return (pos_score.reshape(BATCH, 1), neg_score.reshape(BATCH, 1))


# speedup vs baseline: 1.8522x; 1.1709x over previous
import functools

import jax
import jax.numpy as jnp
from jax import lax
from jax.experimental import pallas as pl
from jax.experimental.pallas import tpu as pltpu
from jax.experimental.pallas import tpu_sc as plsc

BATCH = 4096
EMB = 64
ITEMS = 100000
NC = 2
NS = 16
CPS = EMB // NC
BPT = BATCH // NS
NCH = BPT // 128
GRP = 2
NG = CPS // GRP


def _sc_gather(ue_t, ie_t, beta_row, user_id, last_item, pos_item,
               neg_item):
    mesh = plsc.VectorSubcoreMesh(core_axis_name="c", subcore_axis_name="s")

    @functools.partial(
        pl.kernel,
        mesh=mesh,
        out_type=(
            jax.ShapeDtypeStruct((EMB, BATCH), jnp.float32),
            jax.ShapeDtypeStruct((EMB, BATCH), jnp.float32),
            jax.ShapeDtypeStruct((EMB, BATCH), jnp.float32),
            jax.ShapeDtypeStruct((EMB, BATCH), jnp.float32),
            jax.ShapeDtypeStruct((1, BATCH), jnp.float32),
            jax.ShapeDtypeStruct((1, BATCH), jnp.float32),
        ),
        scratch_types=(
            [pltpu.VMEM_SHARED((ITEMS,), jnp.float32) for _ in range(2 * GRP)]
            + [pltpu.VMEM_SHARED((ITEMS,), jnp.float32) for _ in range(2 * GRP)]
            + [
                pltpu.VMEM_SHARED((ITEMS,), jnp.float32),
                pltpu.VMEM((BPT,), jnp.int32),
                pltpu.VMEM((BPT,), jnp.int32),
                pltpu.VMEM((BPT,), jnp.int32),
                pltpu.VMEM((BPT,), jnp.int32),
                pltpu.VMEM((CPS, BPT), jnp.float32),
                pltpu.VMEM((CPS, BPT), jnp.float32),
                pltpu.VMEM((CPS, BPT), jnp.float32),
                pltpu.VMEM((CPS, BPT), jnp.float32),
                pltpu.VMEM((BPT,), jnp.float32),
                pltpu.VMEM((BPT,), jnp.float32),
                pltpu.SemaphoreType.DMA,
                pltpu.SemaphoreType.DMA,
            ]
        ),
    )
    def k(ue_hbm, ie_hbm, be_hbm, uid_hbm, li_hbm, pi_hbm, ni_hbm,
          gu_hbm, gl_hbm, gp_hbm, gn_hbm, bp_hbm, bn_hbm,
          *refs):
        urows = refs[:2 * GRP]
        irows = refs[2 * GRP:4 * GRP]
        (brow, iu_v, il_v, ip_v, in_v, ou_v, ol_v, op_v, on_v, bp_v, bn_v,
         sem_s, sem_g) = refs[4 * GRP:]
        cid = lax.axis_index("c")
        sid = lax.axis_index("s")
        cbase = cid * CPS
        bbase = sid * BPT

        ci = pltpu.async_copy(uid_hbm.at[pl.ds(bbase, BPT)], iu_v, sem_g)
        ci2 = pltpu.async_copy(li_hbm.at[pl.ds(bbase, BPT)], il_v, sem_g)
        ci3 = pltpu.async_copy(pi_hbm.at[pl.ds(bbase, BPT)], ip_v, sem_g)
        ci4 = pltpu.async_copy(ni_hbm.at[pl.ds(bbase, BPT)], in_v, sem_g)

        def stage(g):
            half = (g % 2) * GRP
            for i in range(GRP):
                @pl.when(sid == i)
                def _():
                    pltpu.async_copy(ue_hbm.at[cbase + g * GRP + i],
                                     urows[half + i], sem_s)

                @pl.when(sid == 8 + i)
                def _():
                    pltpu.async_copy(ie_hbm.at[cbase + g * GRP + i],
                                     irows[half + i], sem_s)

        def stage_wait(g):
            half = (g % 2) * GRP
            for i in range(GRP):
                @pl.when(sid == i)
                def _():
                    pltpu.make_async_copy(ue_hbm.at[cbase],
                                          urows[half + i], sem_s).wait()

                @pl.when(sid == 8 + i)
                def _():
                    pltpu.make_async_copy(ie_hbm.at[cbase],
                                          irows[half + i], sem_s).wait()

        stage(0)

        @pl.when(jnp.logical_and(cid == 0, sid == 15))
        def _():
            pltpu.async_copy(be_hbm.at[0], brow, sem_s).wait()

        ci.wait()
        ci2.wait()
        ci3.wait()
        ci4.wait()

        for g in range(NG):
            half = (g % 2) * GRP
            stage_wait(g)
            plsc.subcore_barrier()
            if g + 1 < NG:
                stage(g + 1)
            cps = []
            for i in range(GRP):
                cc = g * GRP + i
                ub = urows[half + i]
                ib = irows[half + i]
                for ch in range(NCH):
                    sl = pl.ds(ch * 128, 128)
                    cps.append(pltpu.async_copy(
                        ub.at[iu_v.at[sl]], ou_v.at[cc, sl], sem_g))
                    cps.append(pltpu.async_copy(
                        ib.at[il_v.at[sl]], ol_v.at[cc, sl], sem_g))
                    cps.append(pltpu.async_copy(
                        ib.at[ip_v.at[sl]], op_v.at[cc, sl], sem_g))
                    cps.append(pltpu.async_copy(
                        ib.at[in_v.at[sl]], on_v.at[cc, sl], sem_g))
            for c in cps:
                c.wait()
            plsc.subcore_barrier()

        @pl.when(cid == 0)
        def _():
            cbs = []
            for ch in range(NCH):
                sl = pl.ds(ch * 128, 128)
                cbs.append(pltpu.async_copy(
                    brow.at[ip_v.at[sl]], bp_v.at[sl], sem_g))
                cbs.append(pltpu.async_copy(
                    brow.at[in_v.at[sl]], bn_v.at[sl], sem_g))
            for c in cbs:
                c.wait()
            pltpu.sync_copy(bp_v, bp_hbm.at[0, pl.ds(bbase, BPT)])
            pltpu.sync_copy(bn_v, bn_hbm.at[0, pl.ds(bbase, BPT)])

        co1 = pltpu.async_copy(
            ou_v, gu_hbm.at[pl.ds(cbase, CPS), pl.ds(bbase, BPT)], sem_g)
        co2 = pltpu.async_copy(
            ol_v, gl_hbm.at[pl.ds(cbase, CPS), pl.ds(bbase, BPT)], sem_g)
        co3 = pltpu.async_copy(
            op_v, gp_hbm.at[pl.ds(cbase, CPS), pl.ds(bbase, BPT)], sem_g)
        co4 = pltpu.async_copy(
            on_v, gn_hbm.at[pl.ds(cbase, CPS), pl.ds(bbase, BPT)], sem_g)
        co1.wait()
        co2.wait()
        co3.wait()
        co4.wait()

    return k(ue_t, ie_t, beta_row, user_id, last_item, pos_item, neg_item)


def _tc_body(gu_ref, gl_ref, gp_ref, gn_ref, bp_ref, bn_ref, t_ref,
             pos_ref, neg_ref):
    def scale(x):
        n2 = jnp.sum(x * x, axis=0, keepdims=True)
        return 1.0 / jnp.maximum(jnp.sqrt(n2), 1.0)

    gl = gl_ref[...]
    out = gu_ref[...] + t_ref[...] + gl * scale(gl)
    gp = gp_ref[...]
    dp = out - gp * scale(gp)
    gn = gn_ref[...]
    dn = out - gn * scale(gn)
    pos_ref[...] = bp_ref[...] - jnp.sqrt(jnp.sum(dp * dp))
    neg_ref[...] = bn_ref[...] - jnp.sqrt(jnp.sum(dn * dn))


def kernel(user_id, last_item, pos_item, neg_item, user_emb, item_emb, Beta,
           T):
    gu, gl, gp, gn, bp, bn = _sc_gather(
        user_emb.T, item_emb.T, Beta.T,
        user_id.astype(jnp.int32), last_item.astype(jnp.int32),
        pos_item.astype(jnp.int32), neg_item.astype(jnp.int32))
    pos_score, neg_score = pl.pallas_call(
        _tc_body,
        out_shape=(
            jax.ShapeDtypeStruct((1, BATCH), jnp.float32),
            jax.ShapeDtypeStruct((1, BATCH), jnp.float32),
        ),
    )(gu, gl, gp, gn, bp, bn, T.reshape(EMB, 1))
    return (pos_score.reshape(BATCH, 1), neg_score.reshape(BATCH, 1))


# speedup vs baseline: 1.8562x; 1.0022x over previous
import functools

import jax
import jax.numpy as jnp
from jax import lax
from jax.experimental import pallas as pl
from jax.experimental.pallas import tpu as pltpu
from jax.experimental.pallas import tpu_sc as plsc

BATCH = 4096
EMB = 64
ITEMS = 100000
NC = 2
NS = 16
CPS = EMB // NC
BPT = BATCH // NS
NCH = BPT // 128
GRP = 2
NG = CPS // GRP


def _sc_gather(ue_t, ie_t, beta_row, user_id, last_item, pos_item,
               neg_item):
    mesh = plsc.VectorSubcoreMesh(core_axis_name="c", subcore_axis_name="s")

    @functools.partial(
        pl.kernel,
        mesh=mesh,
        out_type=(
            jax.ShapeDtypeStruct((EMB, BATCH), jnp.float32),
            jax.ShapeDtypeStruct((EMB, BATCH), jnp.float32),
            jax.ShapeDtypeStruct((EMB, BATCH), jnp.float32),
            jax.ShapeDtypeStruct((EMB, BATCH), jnp.float32),
            jax.ShapeDtypeStruct((1, BATCH), jnp.float32),
            jax.ShapeDtypeStruct((1, BATCH), jnp.float32),
        ),
        scratch_types=(
            [pltpu.VMEM_SHARED((ITEMS,), jnp.float32) for _ in range(2 * GRP)]
            + [pltpu.VMEM_SHARED((ITEMS,), jnp.float32) for _ in range(2 * GRP)]
            + [
                pltpu.VMEM_SHARED((ITEMS,), jnp.float32),
                pltpu.VMEM((BPT,), jnp.int32),
                pltpu.VMEM((BPT,), jnp.int32),
                pltpu.VMEM((BPT,), jnp.int32),
                pltpu.VMEM((BPT,), jnp.int32),
                pltpu.VMEM((CPS, BPT), jnp.float32),
                pltpu.VMEM((CPS, BPT), jnp.float32),
                pltpu.VMEM((CPS, BPT), jnp.float32),
                pltpu.VMEM((CPS, BPT), jnp.float32),
                pltpu.VMEM((BPT,), jnp.float32),
                pltpu.VMEM((BPT,), jnp.float32),
                pltpu.SemaphoreType.DMA,
                pltpu.SemaphoreType.DMA,
            ]
        ),
    )
    def k(ue_hbm, ie_hbm, be_hbm, uid_hbm, li_hbm, pi_hbm, ni_hbm,
          gu_hbm, gl_hbm, gp_hbm, gn_hbm, bp_hbm, bn_hbm,
          *refs):
        urows = refs[:2 * GRP]
        irows = refs[2 * GRP:4 * GRP]
        (brow, iu_v, il_v, ip_v, in_v, ou_v, ol_v, op_v, on_v, bp_v, bn_v,
         sem_s, sem_g) = refs[4 * GRP:]
        cid = lax.axis_index("c")
        sid = lax.axis_index("s")
        cbase = cid * CPS
        bbase = sid * BPT

        ci = pltpu.async_copy(uid_hbm.at[pl.ds(bbase, BPT)], iu_v, sem_g)
        ci2 = pltpu.async_copy(li_hbm.at[pl.ds(bbase, BPT)], il_v, sem_g)
        ci3 = pltpu.async_copy(pi_hbm.at[pl.ds(bbase, BPT)], ip_v, sem_g)
        ci4 = pltpu.async_copy(ni_hbm.at[pl.ds(bbase, BPT)], in_v, sem_g)

        def stage(g):
            half = (g % 2) * GRP
            for i in range(GRP):
                @pl.when(sid == i)
                def _():
                    pltpu.async_copy(ue_hbm.at[cbase + g * GRP + i],
                                     urows[half + i], sem_s)

                @pl.when(sid == 8 + i)
                def _():
                    pltpu.async_copy(ie_hbm.at[cbase + g * GRP + i],
                                     irows[half + i], sem_s)

        def stage_wait(g):
            half = (g % 2) * GRP
            for i in range(GRP):
                @pl.when(sid == i)
                def _():
                    pltpu.make_async_copy(ue_hbm.at[cbase],
                                          urows[half + i], sem_s).wait()

                @pl.when(sid == 8 + i)
                def _():
                    pltpu.make_async_copy(ie_hbm.at[cbase],
                                          irows[half + i], sem_s).wait()

        stage(0)

        @pl.when(jnp.logical_and(cid == 0, sid == 15))
        def _():
            pltpu.async_copy(be_hbm.at[0], brow, sem_s).wait()

        ci.wait()
        ci2.wait()
        ci3.wait()
        ci4.wait()

        prev = []
        for g in range(NG):
            half = (g % 2) * GRP
            for c in prev:
                c.wait()
            stage_wait(g)
            plsc.subcore_barrier()
            if g + 1 < NG:
                stage(g + 1)
            elif g + 1 == NG:
                pass
            cps = []
            for i in range(GRP):
                cc = g * GRP + i
                ub = urows[half + i]
                ib = irows[half + i]
                for ch in range(NCH):
                    sl = pl.ds(ch * 128, 128)
                    cps.append(pltpu.async_copy(
                        ub.at[iu_v.at[sl]], ou_v.at[cc, sl], sem_g))
                    cps.append(pltpu.async_copy(
                        ib.at[il_v.at[sl]], ol_v.at[cc, sl], sem_g))
                    cps.append(pltpu.async_copy(
                        ib.at[ip_v.at[sl]], op_v.at[cc, sl], sem_g))
                    cps.append(pltpu.async_copy(
                        ib.at[in_v.at[sl]], on_v.at[cc, sl], sem_g))
            prev = cps
        for c in prev:
            c.wait()

        @pl.when(cid == 0)
        def _():
            cbs = []
            for ch in range(NCH):
                sl = pl.ds(ch * 128, 128)
                cbs.append(pltpu.async_copy(
                    brow.at[ip_v.at[sl]], bp_v.at[sl], sem_g))
                cbs.append(pltpu.async_copy(
                    brow.at[in_v.at[sl]], bn_v.at[sl], sem_g))
            for c in cbs:
                c.wait()
            pltpu.sync_copy(bp_v, bp_hbm.at[0, pl.ds(bbase, BPT)])
            pltpu.sync_copy(bn_v, bn_hbm.at[0, pl.ds(bbase, BPT)])

        co1 = pltpu.async_copy(
            ou_v, gu_hbm.at[pl.ds(cbase, CPS), pl.ds(bbase, BPT)], sem_g)
        co2 = pltpu.async_copy(
            ol_v, gl_hbm.at[pl.ds(cbase, CPS), pl.ds(bbase, BPT)], sem_g)
        co3 = pltpu.async_copy(
            op_v, gp_hbm.at[pl.ds(cbase, CPS), pl.ds(bbase, BPT)], sem_g)
        co4 = pltpu.async_copy(
            on_v, gn_hbm.at[pl.ds(cbase, CPS), pl.ds(bbase, BPT)], sem_g)
        co1.wait()
        co2.wait()
        co3.wait()
        co4.wait()

    return k(ue_t, ie_t, beta_row, user_id, last_item, pos_item, neg_item)


def _tc_body(gu_ref, gl_ref, gp_ref, gn_ref, bp_ref, bn_ref, t_ref,
             pos_ref, neg_ref):
    def scale(x):
        n2 = jnp.sum(x * x, axis=0, keepdims=True)
        return 1.0 / jnp.maximum(jnp.sqrt(n2), 1.0)

    gl = gl_ref[...]
    out = gu_ref[...] + t_ref[...] + gl * scale(gl)
    gp = gp_ref[...]
    dp = out - gp * scale(gp)
    gn = gn_ref[...]
    dn = out - gn * scale(gn)
    pos_ref[...] = bp_ref[...] - jnp.sqrt(jnp.sum(dp * dp))
    neg_ref[...] = bn_ref[...] - jnp.sqrt(jnp.sum(dn * dn))


def kernel(user_id, last_item, pos_item, neg_item, user_emb, item_emb, Beta,
           T):
    gu, gl, gp, gn, bp, bn = _sc_gather(
        user_emb.T, item_emb.T, Beta.T,
        user_id.astype(jnp.int32), last_item.astype(jnp.int32),
        pos_item.astype(jnp.int32), neg_item.astype(jnp.int32))
    pos_score, neg_score = pl.pallas_call(
        _tc_body,
        out_shape=(
            jax.ShapeDtypeStruct((1, BATCH), jnp.float32),
            jax.ShapeDtypeStruct((1, BATCH), jnp.float32),
        ),
    )(gu, gl, gp, gn, bp, bn, T.reshape(EMB, 1))
    return (pos_score.reshape(BATCH, 1), neg_score.reshape(BATCH, 1))
